# baseline (speedup 1.0000x reference)
import functools

import jax
import jax.numpy as jnp
from jax import lax
from jax.experimental import pallas as pl
from jax.experimental.pallas import tpu as pltpu
from jax.experimental.pallas import tpu_sc as plsc

NNODES = 10000
D = 128
NC = 2
NS = 16
NW = NC * NS
L = 16

NP = 10240
CH = 128
EPT = 10240
EPAD = NW * EPT
NCHUNK = EPT // CH
ROWS_PER_TILE = NP // NS

GCH = 64
NBUF = 4
GCHUNKS = EPT // GCH
SPH = CH // GCH
NGRP = NBUF // SPH

H0 = 80
H1 = 80
HMAX = max(H0, H1)
assert 16 * (H0 + H1) * CH == EPAD

_mesh = plsc.VectorSubcoreMesh(
    core_axis_name="c", subcore_axis_name="s", num_cores=NC, num_subcores=NS
)


def _deg_kernel(dst3, ones_rows):
    @functools.partial(
        pl.kernel,
        mesh=_mesh,
        out_type=jax.ShapeDtypeStruct((NC, NP, D), jnp.float32),
        scratch_types=[
            pltpu.VMEM((NCHUNK, CH), jnp.int32),
            pltpu.VMEM((CH, D), jnp.float32),
            pltpu.VMEM((CH, D), jnp.float32),
            pltpu.VMEM_SHARED((NP, D), jnp.float32),
        ],
    )
    def k(dst_ref, ones_ref, out_ref, didx, zbuf, buf, dacc):
        cid = lax.axis_index("c")
        sid = lax.axis_index("s")
        wid = sid * NC + cid

        pltpu.sync_copy(dst_ref.at[wid], didx)
        pltpu.sync_copy(ones_ref, buf)
        zero = jnp.zeros((L,), jnp.float32)

        def zrow(r, _):
            for c8 in range(D // L):
                zbuf[r, pl.ds(c8 * L, L)] = zero
            return 0

        lax.fori_loop(0, CH, zrow, 0)
        base = sid * ROWS_PER_TILE
        for j in range(ROWS_PER_TILE // CH):
            pltpu.sync_copy(zbuf, dacc.at[pl.ds(base + j * CH, CH)])
        plsc.subcore_barrier()

        def body(j, _):
            pltpu.sync_copy(buf, dacc.at[didx.at[j]], add=True)
            return 0

        lax.fori_loop(0, NCHUNK, body, 0)
        plsc.subcore_barrier()

        for j in range(ROWS_PER_TILE // CH):
            sl = pl.ds(base + j * CH, CH)
            pltpu.sync_copy(dacc.at[sl], out_ref.at[cid, sl])

    return k(dst3, ones_rows)


def _hop_kernel(t_hbm, src2, dst2):
    @functools.partial(
        pl.kernel,
        mesh=_mesh,
        out_type=jax.ShapeDtypeStruct((NC, NP, D), jnp.float32),
        scratch_types=[
            [pltpu.VMEM((GCH,), jnp.int32) for _ in range(NBUF)],
            pltpu.VMEM((HMAX, CH), jnp.int32),
            pltpu.VMEM((NBUF * GCH, D), jnp.float32),
            pltpu.VMEM_SHARED((NP, D), jnp.float32),
            [pltpu.SemaphoreType.DMA for _ in range(NBUF)],
        ],
    )
    def k(t_ref, src_ref, dst_ref, out_ref, sidx, didx, rows, acc, sem):
        cid = lax.axis_index("c")
        sid = lax.axis_index("s")

        nh = jnp.where(cid == 0, H0, H1)
        hbase = jnp.where(cid == 0, sid * H0, NS * H0 + sid * H1)

        @pl.when(cid == 0)
        def _():
            pltpu.sync_copy(
                dst_ref.at[pl.ds(hbase, H0)], didx.at[pl.ds(0, H0)]
            )

        @pl.when(cid == 1)
        def _():
            pltpu.sync_copy(
                dst_ref.at[pl.ds(hbase, H1)], didx.at[pl.ds(0, H1)]
            )

        zero = jnp.zeros((L,), jnp.float32)

        def zrow(r, _):
            for c8 in range(D // L):
                rows[r, pl.ds(c8 * L, L)] = zero
            return 0

        lax.fori_loop(0, CH, zrow, 0)
        zsrc = rows.at[pl.ds(0, CH)]
        base = sid * ROWS_PER_TILE
        for j in range(ROWS_PER_TILE // CH):
            pltpu.sync_copy(zsrc, acc.at[pl.ds(base + j * CH, CH)])
        plsc.subcore_barrier()

        sbase = SPH * hbase
        nsub = SPH * nh

        def fire(sub, b):
            pltpu.sync_copy(src_ref.at[sbase + sub], sidx[b])
            pltpu.async_copy(
                t_ref.at[sidx[b]], rows.at[pl.ds(b * GCH, GCH)], sem[b]
            )

        def drain(b):
            pltpu.make_async_copy(
                t_ref.at[sidx[b]], rows.at[pl.ds(b * GCH, GCH)], sem[b]
            ).wait()

        for b in range(NBUF):
            fire(b, b)

        def body(j2, _):
            j = NGRP * j2
            for h in range(NGRP):
                qbase = h * SPH
                for q in range(SPH):
                    drain(qbase + q)
                pltpu.sync_copy(
                    rows.at[pl.ds(qbase * GCH, CH)],
                    acc.at[didx.at[j + h]],
                    add=True,
                )
                sub = SPH * (j + h) + NBUF

                @pl.when(sub + SPH - 1 < nsub)
                def _():
                    for q in range(SPH):
                        fire(sub + q, qbase + q)
            return 0

        lax.fori_loop(0, nh // NGRP, body, 0)
        plsc.subcore_barrier()

        for j in range(ROWS_PER_TILE // CH):
            sl = pl.ds(base + j * CH, CH)
            pltpu.sync_copy(acc.at[sl], out_ref.at[cid, sl])

    return k(t_hbm, src2, dst2)


_ROWBLK = 256
_NBLK = NP // _ROWBLK


def _scale_x(x_pad, d0, d1):
    def body(x_ref, d0_ref, d1_ref, t0_ref, dc_ref):
        dcol = d0_ref[...] + d1_ref[...] + 1.0
        dc_ref[...] = dcol[:, :L]
        s = lax.rsqrt(dcol[:, 0:1])
        t0_ref[...] = x_ref[...] * s

    return pl.pallas_call(
        body,
        grid=(_NBLK,),
        in_specs=[
            pl.BlockSpec((_ROWBLK, D), lambda i: (i, 0)),
            pl.BlockSpec((_ROWBLK, D), lambda i: (i, 0)),
            pl.BlockSpec((_ROWBLK, D), lambda i: (i, 0)),
        ],
        out_specs=[
            pl.BlockSpec((_ROWBLK, D), lambda i: (i, 0)),
            pl.BlockSpec((_ROWBLK, L), lambda i: (i, 0)),
        ],
        out_shape=[
            jax.ShapeDtypeStruct((NP, D), jnp.float32),
            jax.ShapeDtypeStruct((NP, L), jnp.float32),
        ],
    )(x_pad, d0, d1)


def _mid_scale(p0, p1, t0, dc):
    def body(p0_ref, p1_ref, t0_ref, dc_ref, t2_ref):
        h = p0_ref[...] + p1_ref[...] + t0_ref[...]
        t2_ref[...] = h / dc_ref[:, 0:1]

    return pl.pallas_call(
        body,
        grid=(_NBLK,),
        in_specs=[
            pl.BlockSpec((_ROWBLK, D), lambda i: (i, 0)),
            pl.BlockSpec((_ROWBLK, D), lambda i: (i, 0)),
            pl.BlockSpec((_ROWBLK, D), lambda i: (i, 0)),
            pl.BlockSpec((_ROWBLK, L), lambda i: (i, 0)),
        ],
        out_specs=pl.BlockSpec((_ROWBLK, D), lambda i: (i, 0)),
        out_shape=jax.ShapeDtypeStruct((NP, D), jnp.float32),
    )(p0, p1, t0, dc)


def _final(q0, q1, t2, dc, W, b2):
    def body(q0_ref, q1_ref, t2_ref, dc_ref, w_ref, b_ref, o_ref):
        h = (q0_ref[...] + q1_ref[...] + t2_ref[...]) * lax.rsqrt(
            dc_ref[:, 0:1]
        )
        logits = (
            lax.dot_general(
                h,
                w_ref[...],
                (((1,), (1,)), ((), ())),
                preferred_element_type=jnp.float32,
            )
            + b_ref[...]
        )
        m = jnp.max(logits, axis=1, keepdims=True)
        e = jnp.exp(logits - m)
        lse = jnp.log(jnp.sum(e, axis=1, keepdims=True)) + m
        o_ref[...] = logits - lse

    return pl.pallas_call(
        body,
        grid=(_NBLK,),
        in_specs=[
            pl.BlockSpec((_ROWBLK, D), lambda i: (i, 0)),
            pl.BlockSpec((_ROWBLK, D), lambda i: (i, 0)),
            pl.BlockSpec((_ROWBLK, D), lambda i: (i, 0)),
            pl.BlockSpec((_ROWBLK, L), lambda i: (i, 0)),
            pl.BlockSpec((D, D), lambda i: (0, 0)),
            pl.BlockSpec((1, D), lambda i: (0, 0)),
        ],
        out_specs=pl.BlockSpec((_ROWBLK, D), lambda i: (i, 0)),
        out_shape=jax.ShapeDtypeStruct((NP, D), jnp.float32),
    )(q0, q1, t2, dc, W, b2)


def kernel(x, edge_index, W, b):
    n, d = x.shape
    e = edge_index.shape[1]
    pad_e = EPAD - e

    src = edge_index[0]
    dst = edge_index[1]
    fill = jnp.full((pad_e,), n, dtype=jnp.int32)
    src_pad = jnp.concatenate([src, fill])
    dst_pad = jnp.concatenate([dst, fill])
    src2 = src_pad.reshape(EPAD // GCH, GCH)
    dst2 = dst_pad.reshape(EPAD // CH, CH)
    dst3 = dst_pad.reshape(NW, NCHUNK, CH)
    x_pad = jnp.concatenate(
        [x, jnp.zeros((NP - n, d), dtype=x.dtype)], axis=0
    )

    ones_rows = jnp.ones((CH, D), jnp.float32)
    dparts = _deg_kernel(dst3, ones_rows)
    t0, dc = _scale_x(x_pad, dparts[0], dparts[1])
    p = _hop_kernel(t0, src2, dst2)
    t2 = _mid_scale(p[0], p[1], t0, dc)
    q = _hop_kernel(t2, src2, dst2)
    out = _final(q[0], q[1], t2, dc, W, b.reshape(1, D))
    return out[:n]


# speedup vs baseline: 1.2430x; 1.2430x over previous
import functools

import jax
import jax.numpy as jnp
from jax import lax
from jax.experimental import pallas as pl
from jax.experimental.pallas import tpu as pltpu
from jax.experimental.pallas import tpu_sc as plsc

NNODES = 10000
D = 128
NC = 2
NS = 16
NW = NC * NS
L = 16

NP = 10240
CH = 128
EPT = 10240
EPAD = NW * EPT
NCHUNK = EPT // CH
ROWS_PER_TILE = NP // NS

GCH = 64
NBUF = 4
GCHUNKS = EPT // GCH
SPH = CH // GCH
NGRP = NBUF // SPH

_mesh = plsc.VectorSubcoreMesh(
    core_axis_name="c", subcore_axis_name="s", num_cores=NC, num_subcores=NS
)


def _deg_kernel(dst3, ones_rows):
    @functools.partial(
        pl.kernel,
        mesh=_mesh,
        out_type=jax.ShapeDtypeStruct((NC, NP, D), jnp.float32),
        scratch_types=[
            pltpu.VMEM((NCHUNK, CH), jnp.int32),
            pltpu.VMEM((CH, D), jnp.float32),
            pltpu.VMEM((CH, D), jnp.float32),
            pltpu.VMEM_SHARED((NP, D), jnp.float32),
        ],
    )
    def k(dst_ref, ones_ref, out_ref, didx, zbuf, buf, dacc):
        cid = lax.axis_index("c")
        sid = lax.axis_index("s")
        wid = sid * NC + cid

        pltpu.sync_copy(dst_ref.at[wid], didx)
        pltpu.sync_copy(ones_ref, buf)
        zero = jnp.zeros((L,), jnp.float32)

        def zrow(r, _):
            for c8 in range(D // L):
                zbuf[r, pl.ds(c8 * L, L)] = zero
            return 0

        lax.fori_loop(0, CH, zrow, 0)
        base = sid * ROWS_PER_TILE
        for j in range(ROWS_PER_TILE // CH):
            pltpu.sync_copy(zbuf, dacc.at[pl.ds(base + j * CH, CH)])
        plsc.subcore_barrier()

        def body(j, _):
            pltpu.sync_copy(buf, dacc.at[didx.at[j]], add=True)
            return 0

        lax.fori_loop(0, NCHUNK, body, 0)
        plsc.subcore_barrier()

        for j in range(ROWS_PER_TILE // CH):
            sl = pl.ds(base + j * CH, CH)
            pltpu.sync_copy(dacc.at[sl], out_ref.at[cid, sl])

    return k(dst3, ones_rows)


def _hop_kernel(t_hbm, src3, dst3):
    @functools.partial(
        pl.kernel,
        mesh=_mesh,
        out_type=jax.ShapeDtypeStruct((NC, NP, D), jnp.float32),
        scratch_types=[
            [pltpu.VMEM((GCH,), jnp.int32) for _ in range(NBUF)],
            pltpu.VMEM((NCHUNK, CH), jnp.int32),
            pltpu.VMEM((NBUF * GCH, D), jnp.float32),
            pltpu.VMEM_SHARED((NP, D), jnp.float32),
            [pltpu.SemaphoreType.DMA for _ in range(NBUF)],
        ],
    )
    def k(t_ref, src_ref, dst_ref, out_ref, sidx, didx, rows, acc, sem):
        cid = lax.axis_index("c")
        sid = lax.axis_index("s")
        wid = sid * NC + cid

        pltpu.sync_copy(dst_ref.at[wid], didx)

        zero = jnp.zeros((L,), jnp.float32)

        def zrow(r, _):
            for c8 in range(D // L):
                rows[r, pl.ds(c8 * L, L)] = zero
            return 0

        lax.fori_loop(0, CH, zrow, 0)
        zsrc = rows.at[pl.ds(0, CH)]
        base = sid * ROWS_PER_TILE
        for j in range(ROWS_PER_TILE // CH):
            pltpu.sync_copy(zsrc, acc.at[pl.ds(base + j * CH, CH)])
        plsc.subcore_barrier()

        def fire(sub, b):
            pltpu.sync_copy(src_ref.at[wid, sub], sidx[b])
            pltpu.async_copy(
                t_ref.at[sidx[b]], rows.at[pl.ds(b * GCH, GCH)], sem[b]
            )

        def drain(b):
            pltpu.make_async_copy(
                t_ref.at[sidx[b]], rows.at[pl.ds(b * GCH, GCH)], sem[b]
            ).wait()

        for b in range(NBUF):
            fire(b, b)

        def body(j2, _):
            j = NGRP * j2
            for h in range(NGRP):
                qbase = h * SPH
                for q in range(SPH):
                    drain(qbase + q)
                pltpu.sync_copy(
                    rows.at[pl.ds(qbase * GCH, CH)],
                    acc.at[didx.at[j + h]],
                    add=True,
                )
                sub = SPH * (j + h) + NBUF

                @pl.when(sub + SPH - 1 < GCHUNKS)
                def _():
                    for q in range(SPH):
                        fire(sub + q, qbase + q)
            return 0

        lax.fori_loop(0, NCHUNK // NGRP, body, 0)
        plsc.subcore_barrier()

        for j in range(ROWS_PER_TILE // CH):
            sl = pl.ds(base + j * CH, CH)
            pltpu.sync_copy(acc.at[sl], out_ref.at[cid, sl])

    return k(t_hbm, src3, dst3)


_ROWBLK = 256
_NBLK = NP // _ROWBLK


def _scale_x(x_pad, d0, d1):
    def body(x_ref, d0_ref, d1_ref, t0_ref, dc_ref):
        dcol = d0_ref[...] + d1_ref[...] + 1.0
        dc_ref[...] = dcol[:, :L]
        s = lax.rsqrt(dcol[:, 0:1])
        t0_ref[...] = x_ref[...] * s

    return pl.pallas_call(
        body,
        grid=(_NBLK,),
        in_specs=[
            pl.BlockSpec((_ROWBLK, D), lambda i: (i, 0)),
            pl.BlockSpec((_ROWBLK, D), lambda i: (i, 0)),
            pl.BlockSpec((_ROWBLK, D), lambda i: (i, 0)),
        ],
        out_specs=[
            pl.BlockSpec((_ROWBLK, D), lambda i: (i, 0)),
            pl.BlockSpec((_ROWBLK, L), lambda i: (i, 0)),
        ],
        out_shape=[
            jax.ShapeDtypeStruct((NP, D), jnp.float32),
            jax.ShapeDtypeStruct((NP, L), jnp.float32),
        ],
    )(x_pad, d0, d1)


def _mid_scale(p0, p1, t0, dc):
    def body(p0_ref, p1_ref, t0_ref, dc_ref, t2_ref):
        h = p0_ref[...] + p1_ref[...] + t0_ref[...]
        t2_ref[...] = h / dc_ref[:, 0:1]

    return pl.pallas_call(
        body,
        grid=(_NBLK,),
        in_specs=[
            pl.BlockSpec((_ROWBLK, D), lambda i: (i, 0)),
            pl.BlockSpec((_ROWBLK, D), lambda i: (i, 0)),
            pl.BlockSpec((_ROWBLK, D), lambda i: (i, 0)),
            pl.BlockSpec((_ROWBLK, L), lambda i: (i, 0)),
        ],
        out_specs=pl.BlockSpec((_ROWBLK, D), lambda i: (i, 0)),
        out_shape=jax.ShapeDtypeStruct((NP, D), jnp.float32),
    )(p0, p1, t0, dc)


def _final(q0, q1, t2, dc, W, b2):
    def body(q0_ref, q1_ref, t2_ref, dc_ref, w_ref, b_ref, o_ref):
        h = (q0_ref[...] + q1_ref[...] + t2_ref[...]) * lax.rsqrt(
            dc_ref[:, 0:1]
        )
        logits = (
            lax.dot_general(
                h,
                w_ref[...],
                (((1,), (1,)), ((), ())),
                preferred_element_type=jnp.float32,
            )
            + b_ref[...]
        )
        m = jnp.max(logits, axis=1, keepdims=True)
        e = jnp.exp(logits - m)
        lse = jnp.log(jnp.sum(e, axis=1, keepdims=True)) + m
        o_ref[...] = logits - lse

    return pl.pallas_call(
        body,
        grid=(_NBLK,),
        in_specs=[
            pl.BlockSpec((_ROWBLK, D), lambda i: (i, 0)),
            pl.BlockSpec((_ROWBLK, D), lambda i: (i, 0)),
            pl.BlockSpec((_ROWBLK, D), lambda i: (i, 0)),
            pl.BlockSpec((_ROWBLK, L), lambda i: (i, 0)),
            pl.BlockSpec((D, D), lambda i: (0, 0)),
            pl.BlockSpec((1, D), lambda i: (0, 0)),
        ],
        out_specs=pl.BlockSpec((_ROWBLK, D), lambda i: (i, 0)),
        out_shape=jax.ShapeDtypeStruct((NP, D), jnp.float32),
    )(q0, q1, t2, dc, W, b2)


def kernel(x, edge_index, W, b):
    n, d = x.shape
    e = edge_index.shape[1]
    pad_e = EPAD - e

    src = edge_index[0]
    dst = edge_index[1]
    fill = jnp.full((pad_e,), n, dtype=jnp.int32)
    src_pad = jnp.concatenate([src, fill])
    dst_pad = jnp.concatenate([dst, fill])
    src3 = src_pad.reshape(NW, GCHUNKS, GCH)
    dst3 = dst_pad.reshape(NW, NCHUNK, CH)
    x_pad = jnp.concatenate(
        [x, jnp.zeros((NP - n, d), dtype=x.dtype)], axis=0
    )

    ones_rows = jnp.ones((CH, D), jnp.float32)
    dparts = _deg_kernel(dst3, ones_rows)
    t0, dc = _scale_x(x_pad, dparts[0], dparts[1])
    p = _hop_kernel(t0, src3, dst3)
    t2 = _mid_scale(p[0], p[1], t0, dc)
    q = _hop_kernel(t2, src3, dst3)
    out = _final(q[0], q[1], t2, dc, W, b.reshape(1, D))
    return out[:n]


# speedup vs baseline: 1.2446x; 1.0013x over previous
import functools

import jax
import jax.numpy as jnp
from jax import lax
from jax.experimental import pallas as pl
from jax.experimental.pallas import tpu as pltpu
from jax.experimental.pallas import tpu_sc as plsc

NNODES = 10000
D = 128
NC = 2
NS = 16
NW = NC * NS
L = 16

NP = 10240
CH = 128
EPT = 10240
EPAD = NW * EPT
NCHUNK = EPT // CH
ROWS_PER_TILE = NP // NS

GCH = 64
NBUF = 4
GCHUNKS = EPT // GCH
SPH = CH // GCH
NGRP = NBUF // SPH

_mesh = plsc.VectorSubcoreMesh(
    core_axis_name="c", subcore_axis_name="s", num_cores=NC, num_subcores=NS
)


def _deg_kernel(dst3, ones_rows):
    @functools.partial(
        pl.kernel,
        mesh=_mesh,
        out_type=jax.ShapeDtypeStruct((NC, NP, D), jnp.float32),
        scratch_types=[
            pltpu.VMEM((NCHUNK, CH), jnp.int32),
            pltpu.VMEM((CH, D), jnp.float32),
            pltpu.VMEM((CH, D), jnp.float32),
            pltpu.VMEM_SHARED((NP, D), jnp.float32),
            pltpu.SemaphoreType.DMA,
            pltpu.SemaphoreType.DMA,
        ],
    )
    def k(dst_ref, ones_ref, out_ref, didx, zbuf, buf, dacc, sem_a, sem_b):
        cid = lax.axis_index("c")
        sid = lax.axis_index("s")
        wid = sid * NC + cid

        pltpu.sync_copy(dst_ref.at[wid], didx)
        pltpu.sync_copy(ones_ref, buf)
        zero = jnp.zeros((L,), jnp.float32)

        def zrow(r, _):
            for c8 in range(D // L):
                zbuf[r, pl.ds(c8 * L, L)] = zero
            return 0

        lax.fori_loop(0, CH, zrow, 0)
        base = sid * ROWS_PER_TILE
        for j in range(ROWS_PER_TILE // CH):
            pltpu.sync_copy(zbuf, dacc.at[pl.ds(base + j * CH, CH)])
        plsc.subcore_barrier()

        pltpu.async_copy(buf, dacc.at[didx.at[0]], sem_a, add=True)
        pltpu.async_copy(buf, dacc.at[didx.at[1]], sem_b, add=True)

        def body(j2, _):
            j = 2 * j2
            pltpu.make_async_copy(buf, dacc.at[didx.at[j]], sem_a).wait()

            @pl.when(j + 2 < NCHUNK)
            def _():
                pltpu.async_copy(
                    buf, dacc.at[didx.at[j + 2]], sem_a, add=True
                )

            pltpu.make_async_copy(
                buf, dacc.at[didx.at[j + 1]], sem_b
            ).wait()

            @pl.when(j + 3 < NCHUNK)
            def _():
                pltpu.async_copy(
                    buf, dacc.at[didx.at[j + 3]], sem_b, add=True
                )

            return 0

        lax.fori_loop(0, NCHUNK // 2, body, 0)
        plsc.subcore_barrier()

        for j in range(ROWS_PER_TILE // CH):
            sl = pl.ds(base + j * CH, CH)
            pltpu.sync_copy(dacc.at[sl], out_ref.at[cid, sl])

    return k(dst3, ones_rows)


def _hop_kernel(t_hbm, src3, dst3):
    @functools.partial(
        pl.kernel,
        mesh=_mesh,
        out_type=jax.ShapeDtypeStruct((NC, NP, D), jnp.float32),
        scratch_types=[
            [pltpu.VMEM((GCH,), jnp.int32) for _ in range(NBUF)],
            pltpu.VMEM((NCHUNK, CH), jnp.int32),
            pltpu.VMEM((NBUF * GCH, D), jnp.float32),
            pltpu.VMEM_SHARED((NP, D), jnp.float32),
            [pltpu.SemaphoreType.DMA for _ in range(NBUF)],
        ],
    )
    def k(t_ref, src_ref, dst_ref, out_ref, sidx, didx, rows, acc, sem):
        cid = lax.axis_index("c")
        sid = lax.axis_index("s")
        wid = sid * NC + cid

        pltpu.sync_copy(dst_ref.at[wid], didx)

        zero = jnp.zeros((L,), jnp.float32)

        def zrow(r, _):
            for c8 in range(D // L):
                rows[r, pl.ds(c8 * L, L)] = zero
            return 0

        lax.fori_loop(0, CH, zrow, 0)
        zsrc = rows.at[pl.ds(0, CH)]
        base = sid * ROWS_PER_TILE
        for j in range(ROWS_PER_TILE // CH):
            pltpu.sync_copy(zsrc, acc.at[pl.ds(base + j * CH, CH)])
        plsc.subcore_barrier()

        def fire(sub, b):
            pltpu.sync_copy(src_ref.at[wid, sub], sidx[b])
            pltpu.async_copy(
                t_ref.at[sidx[b]], rows.at[pl.ds(b * GCH, GCH)], sem[b]
            )

        def drain(b):
            pltpu.make_async_copy(
                t_ref.at[sidx[b]], rows.at[pl.ds(b * GCH, GCH)], sem[b]
            ).wait()

        for b in range(NBUF):
            fire(b, b)

        def body(j2, _):
            j = NGRP * j2
            for h in range(NGRP):
                qbase = h * SPH
                for q in range(SPH):
                    drain(qbase + q)
                pltpu.sync_copy(
                    rows.at[pl.ds(qbase * GCH, CH)],
                    acc.at[didx.at[j + h]],
                    add=True,
                )
                sub = SPH * (j + h) + NBUF

                @pl.when(sub + SPH - 1 < GCHUNKS)
                def _():
                    for q in range(SPH):
                        fire(sub + q, qbase + q)
            return 0

        lax.fori_loop(0, NCHUNK // NGRP, body, 0)
        plsc.subcore_barrier()

        for j in range(ROWS_PER_TILE // CH):
            sl = pl.ds(base + j * CH, CH)
            pltpu.sync_copy(acc.at[sl], out_ref.at[cid, sl])

    return k(t_hbm, src3, dst3)


_ROWBLK = 256
_NBLK = NP // _ROWBLK


def _scale_x(x_pad, d0, d1):
    def body(x_ref, d0_ref, d1_ref, t0_ref, dc_ref):
        dcol = d0_ref[...] + d1_ref[...] + 1.0
        dc_ref[...] = dcol[:, :L]
        s = lax.rsqrt(dcol[:, 0:1])
        t0_ref[...] = x_ref[...] * s

    return pl.pallas_call(
        body,
        grid=(_NBLK,),
        in_specs=[
            pl.BlockSpec((_ROWBLK, D), lambda i: (i, 0)),
            pl.BlockSpec((_ROWBLK, D), lambda i: (i, 0)),
            pl.BlockSpec((_ROWBLK, D), lambda i: (i, 0)),
        ],
        out_specs=[
            pl.BlockSpec((_ROWBLK, D), lambda i: (i, 0)),
            pl.BlockSpec((_ROWBLK, L), lambda i: (i, 0)),
        ],
        out_shape=[
            jax.ShapeDtypeStruct((NP, D), jnp.float32),
            jax.ShapeDtypeStruct((NP, L), jnp.float32),
        ],
    )(x_pad, d0, d1)


def _mid_scale(p0, p1, t0, dc):
    def body(p0_ref, p1_ref, t0_ref, dc_ref, t2_ref):
        h = p0_ref[...] + p1_ref[...] + t0_ref[...]
        t2_ref[...] = h / dc_ref[:, 0:1]

    return pl.pallas_call(
        body,
        grid=(_NBLK,),
        in_specs=[
            pl.BlockSpec((_ROWBLK, D), lambda i: (i, 0)),
            pl.BlockSpec((_ROWBLK, D), lambda i: (i, 0)),
            pl.BlockSpec((_ROWBLK, D), lambda i: (i, 0)),
            pl.BlockSpec((_ROWBLK, L), lambda i: (i, 0)),
        ],
        out_specs=pl.BlockSpec((_ROWBLK, D), lambda i: (i, 0)),
        out_shape=jax.ShapeDtypeStruct((NP, D), jnp.float32),
    )(p0, p1, t0, dc)


def _final(q0, q1, t2, dc, W, b2):
    def body(q0_ref, q1_ref, t2_ref, dc_ref, w_ref, b_ref, o_ref):
        h = (q0_ref[...] + q1_ref[...] + t2_ref[...]) * lax.rsqrt(
            dc_ref[:, 0:1]
        )
        logits = (
            lax.dot_general(
                h,
                w_ref[...],
                (((1,), (1,)), ((), ())),
                preferred_element_type=jnp.float32,
            )
            + b_ref[...]
        )
        m = jnp.max(logits, axis=1, keepdims=True)
        e = jnp.exp(logits - m)
        lse = jnp.log(jnp.sum(e, axis=1, keepdims=True)) + m
        o_ref[...] = logits - lse

    return pl.pallas_call(
        body,
        grid=(_NBLK,),
        in_specs=[
            pl.BlockSpec((_ROWBLK, D), lambda i: (i, 0)),
            pl.BlockSpec((_ROWBLK, D), lambda i: (i, 0)),
            pl.BlockSpec((_ROWBLK, D), lambda i: (i, 0)),
            pl.BlockSpec((_ROWBLK, L), lambda i: (i, 0)),
            pl.BlockSpec((D, D), lambda i: (0, 0)),
            pl.BlockSpec((1, D), lambda i: (0, 0)),
        ],
        out_specs=pl.BlockSpec((_ROWBLK, D), lambda i: (i, 0)),
        out_shape=jax.ShapeDtypeStruct((NP, D), jnp.float32),
    )(q0, q1, t2, dc, W, b2)


def kernel(x, edge_index, W, b):
    n, d = x.shape
    e = edge_index.shape[1]
    pad_e = EPAD - e

    src = edge_index[0]
    dst = edge_index[1]
    fill = jnp.full((pad_e,), n, dtype=jnp.int32)
    src_pad = jnp.concatenate([src, fill])
    dst_pad = jnp.concatenate([dst, fill])
    src3 = src_pad.reshape(NW, GCHUNKS, GCH)
    dst3 = dst_pad.reshape(NW, NCHUNK, CH)
    x_pad = jnp.concatenate(
        [x, jnp.zeros((NP - n, d), dtype=x.dtype)], axis=0
    )

    ones_rows = jnp.ones((CH, D), jnp.float32)
    dparts = _deg_kernel(dst3, ones_rows)
    t0, dc = _scale_x(x_pad, dparts[0], dparts[1])
    p = _hop_kernel(t0, src3, dst3)
    t2 = _mid_scale(p[0], p[1], t0, dc)
    q = _hop_kernel(t2, src3, dst3)
    out = _final(q[0], q[1], t2, dc, W, b.reshape(1, D))
    return out[:n]
